# Initial kernel scaffold; baseline (speedup 1.0000x reference)
import jax
import jax.numpy as jnp
from jax.experimental import pallas as pl


def kernel(x, edge_index, W1, b1, W2, b2):
    raise NotImplementedError("write your pallas kernel here")


# speedup vs baseline: 2.9452x; 2.9452x over previous
import functools

import jax
import jax.numpy as jnp
from jax import lax
from jax.experimental import pallas as pl
from jax.experimental.pallas import tpu as pltpu
from jax.experimental.pallas import tpu_sc as plsc

N_NODES = 10000
N_EDGES = 320000
D = 128

NC = 2
NS = 16
NW = NC * NS
CH = 128
NCHUNK = 2560
E_PAD = NCHUNK * CH
CPW = NCHUNK // NW
CPT = NCHUNK // NS
NACC = 10240
TRASH = N_NODES
RPT = NACC // NS
NPT = NACC // NW

_mesh = plsc.VectorSubcoreMesh(core_axis_name="c", subcore_axis_name="s")


def _fill_zero_row(zrow):
    for u in range(8):
        zrow[pl.ds(u * 16, 16)] = jnp.zeros((16,), jnp.float32)


def _sc_deg_kernel(rowc_hbm, dinv_hbm, rowv, ones, zrow, degb, dinvb, rbuf,
                   deg_sh, sem):
    cid = lax.axis_index("c")
    sid = lax.axis_index("s")
    for u in range(8):
        ones[pl.ds(u * 16, 16)] = jnp.ones((16,), jnp.float32)
    _fill_zero_row(zrow)
    pltpu.sync_copy(rowc_hbm.at[pl.ds(sid * CPT, CPT)], rowv)
    for k in range(RPT // 128):
        pltpu.sync_copy(zrow, deg_sh.at[pl.ds(sid * RPT + k * 128, 128)])
    plsc.subcore_barrier()

    def body(j, _):
        pltpu.sync_copy(ones, deg_sh.at[rowv.at[j]], add=True)
        return 0
    lax.fori_loop(0, CPT, body, 0)
    plsc.subcore_barrier()

    node_base = cid * (NACC // NC) + sid * NPT
    pltpu.sync_copy(deg_sh.at[pl.ds(node_base, NPT)], degb)
    for k in range(NPT // 16):
        v = degb[pl.ds(k * 16, 16)]
        dinvb[pl.ds(k * 16, 16)] = jnp.where(
            v == 0.0, jnp.zeros((16,), jnp.float32), 1.0 / v)

    def blk_body(blk, _):
        for k4 in range(4):
            v = dinvb[pl.ds(blk * 64 + k4 * 16, 16)]
            for lane in range(16):
                idx = jnp.full((16,), lane, jnp.int32)
                g = v.at[idx].get(mode="promise_in_bounds")
                for u in range(8):
                    rbuf[pl.ds((k4 * 16 + lane) * D + u * 16, 16)] = g
        pltpu.sync_copy(
            rbuf, dinv_hbm.at[pl.ds((node_base + blk * 64) * D, 64 * D)])
        return 0
    lax.fori_loop(0, NPT // 64, blk_body, 0)


@functools.partial(
    pl.kernel,
    out_type=jax.ShapeDtypeStruct((NACC * D,), jnp.float32),
    mesh=_mesh,
    scratch_types=[
        pltpu.VMEM((CPT, CH), jnp.int32),
        pltpu.VMEM((CH,), jnp.float32),
        pltpu.VMEM((CH,), jnp.float32),
        pltpu.VMEM((NPT,), jnp.float32),
        pltpu.VMEM((NPT,), jnp.float32),
        pltpu.VMEM((64 * D,), jnp.float32),
        pltpu.VMEM_SHARED((NACC,), jnp.float32),
        pltpu.SemaphoreType.DMA,
    ],
)
def _sc_deg(rowc_hbm, dinv_hbm, *rest):
    _sc_deg_kernel(rowc_hbm, dinv_hbm, *rest)


def _sc_seg_kernel(h_hbm, colc_hbm, rowc_hbm, z_hbm, out_hbm, colv, rowv, rows,
                   acc_sh, sem):
    cid = lax.axis_index("c")
    sid = lax.axis_index("s")
    wid = cid * NS + sid
    pltpu.sync_copy(colc_hbm.at[pl.ds(wid * CPW, CPW)], colv)
    pltpu.sync_copy(rowc_hbm.at[pl.ds(wid * CPW, CPW)], rowv)
    for k in range(RPT // 128):
        pltpu.sync_copy(z_hbm, acc_sh.at[pl.ds(sid * RPT + k * 128, 128)])
    plsc.subcore_barrier()

    def body(j, _):
        pltpu.async_copy(h_hbm.at[colv.at[j]], rows, sem).wait()
        pltpu.sync_copy(rows, acc_sh.at[rowv.at[j]], add=True)
        return 0
    lax.fori_loop(0, CPW, body, 0)
    plsc.subcore_barrier()
    pltpu.sync_copy(acc_sh.at[pl.ds(sid * RPT, RPT)],
                    out_hbm.at[cid, pl.ds(sid * RPT, RPT)])


@functools.partial(
    pl.kernel,
    out_type=jax.ShapeDtypeStruct((NC, NACC, D), jnp.float32),
    mesh=_mesh,
    scratch_types=[
        pltpu.VMEM((CPW, CH), jnp.int32),
        pltpu.VMEM((CPW, CH), jnp.int32),
        pltpu.VMEM((CH, D), jnp.float32),
        pltpu.VMEM_SHARED((NACC, D), jnp.float32),
        pltpu.SemaphoreType.DMA,
    ],
)
def _sc_seg(h_hbm, colc_hbm, rowc_hbm, z_hbm, *rest):
    _sc_seg_kernel(h_hbm, colc_hbm, rowc_hbm, z_hbm, *rest)


BLK = 2000


def _tc_in_kernel(x_ref, w_ref, b_ref, o_ref):
    o_ref[...] = lax.dot_general(
        x_ref[...], w_ref[...], (((1,), (1,)), ((), ())),
        preferred_element_type=jnp.float32) + b_ref[...]


def _tc_mid_kernel(p0_ref, p1_ref, dv_ref, w_ref, b_ref, o_ref):
    s = (p0_ref[0] + p1_ref[0]) * dv_ref[...]
    e = jnp.where(s > 0.0, s, jnp.exp(jnp.minimum(s, 0.0)) - 1.0)
    o_ref[...] = lax.dot_general(
        e, w_ref[...], (((1,), (1,)), ((), ())),
        preferred_element_type=jnp.float32) + b_ref[...]


def _tc_fin_kernel(p0_ref, p1_ref, dv_ref, o_ref):
    o_ref[...] = (p0_ref[0] + p1_ref[0]) * dv_ref[...]


def _tc_in(x, w, b2d):
    return pl.pallas_call(
        _tc_in_kernel,
        grid=(N_NODES // BLK,),
        in_specs=[
            pl.BlockSpec((BLK, D), lambda i: (i, 0)),
            pl.BlockSpec((D, D), lambda i: (0, 0)),
            pl.BlockSpec((1, D), lambda i: (0, 0)),
        ],
        out_specs=pl.BlockSpec((BLK, D), lambda i: (i, 0)),
        out_shape=jax.ShapeDtypeStruct((N_NODES, D), jnp.float32),
    )(x, w, b2d)


def _tc_mid(p, dinv, w, b2d):
    return pl.pallas_call(
        _tc_mid_kernel,
        grid=(N_NODES // BLK,),
        in_specs=[
            pl.BlockSpec((1, BLK, D), lambda i: (0, i, 0)),
            pl.BlockSpec((1, BLK, D), lambda i: (1, i, 0)),
            pl.BlockSpec((BLK, D), lambda i: (i, 0)),
            pl.BlockSpec((D, D), lambda i: (0, 0)),
            pl.BlockSpec((1, D), lambda i: (0, 0)),
        ],
        out_specs=pl.BlockSpec((BLK, D), lambda i: (i, 0)),
        out_shape=jax.ShapeDtypeStruct((N_NODES, D), jnp.float32),
    )(p, p, dinv, w, b2d)


def _tc_fin(p, dinv):
    return pl.pallas_call(
        _tc_fin_kernel,
        grid=(N_NODES // BLK,),
        in_specs=[
            pl.BlockSpec((1, BLK, D), lambda i: (0, i, 0)),
            pl.BlockSpec((1, BLK, D), lambda i: (1, i, 0)),
            pl.BlockSpec((BLK, D), lambda i: (i, 0)),
        ],
        out_specs=pl.BlockSpec((BLK, D), lambda i: (i, 0)),
        out_shape=jax.ShapeDtypeStruct((N_NODES, D), jnp.float32),
    )(p, p, dinv)


def kernel(x, edge_index, W1, b1, W2, b2):
    row = edge_index[0].astype(jnp.int32)
    col = edge_index[1].astype(jnp.int32)
    pad = E_PAD - N_EDGES
    rowc = jnp.concatenate(
        [row, jnp.full((pad,), TRASH, jnp.int32)]).reshape(NCHUNK, CH)
    colc = jnp.concatenate(
        [col, jnp.zeros((pad,), jnp.int32)]).reshape(NCHUNK, CH)
    b1r = b1.reshape(1, D)
    b2r = b2.reshape(1, D)

    zblk = jnp.zeros((128, D), jnp.float32)

    dinv = _sc_deg(rowc).reshape(NACC, D)
    h1 = _tc_in(x, W1, b1r)
    p1 = _sc_seg(h1, colc, rowc, zblk)
    h2 = _tc_mid(p1, dinv, W2, b2r)
    p2 = _sc_seg(h2, colc, rowc, zblk)
    return _tc_fin(p2, dinv)


# speedup vs baseline: 3.4505x; 1.1716x over previous
import functools

import jax
import jax.numpy as jnp
from jax import lax
from jax.experimental import pallas as pl
from jax.experimental.pallas import tpu as pltpu
from jax.experimental.pallas import tpu_sc as plsc

N_NODES = 10000
N_EDGES = 320000
D = 128

NC = 2
NS = 16
NW = NC * NS
CH = 128
NCHUNK = 2560
E_PAD = NCHUNK * CH
CPW = NCHUNK // NW
HALF = CPW // 2
CPT = NCHUNK // NS
NACC = 10240
TRASH = N_NODES
RPT = NACC // NS
NPT = NACC // NW

_mesh = plsc.VectorSubcoreMesh(core_axis_name="c", subcore_axis_name="s")


def _fill_zero_row(zrow):
    for u in range(8):
        zrow[pl.ds(u * 16, 16)] = jnp.zeros((16,), jnp.float32)


def _sc_deg_kernel(rowc_hbm, dinv_hbm, rowv, ones, zrow, degb, dinvb, rbuf,
                   deg_sh, sem):
    cid = lax.axis_index("c")
    sid = lax.axis_index("s")
    for u in range(8):
        ones[pl.ds(u * 16, 16)] = jnp.ones((16,), jnp.float32)
    _fill_zero_row(zrow)
    pltpu.sync_copy(rowc_hbm.at[pl.ds(sid * CPT, CPT)], rowv)
    for k in range(RPT // 128):
        pltpu.sync_copy(zrow, deg_sh.at[pl.ds(sid * RPT + k * 128, 128)])
    plsc.subcore_barrier()

    def body(j, _):
        pltpu.sync_copy(ones, deg_sh.at[rowv.at[j]], add=True)
        return 0
    lax.fori_loop(0, CPT, body, 0)
    plsc.subcore_barrier()

    node_base = cid * (NACC // NC) + sid * NPT
    pltpu.sync_copy(deg_sh.at[pl.ds(node_base, NPT)], degb)
    for k in range(NPT // 16):
        v = degb[pl.ds(k * 16, 16)]
        dinvb[pl.ds(k * 16, 16)] = jnp.where(
            v == 0.0, jnp.zeros((16,), jnp.float32), 1.0 / v)

    def blk_body(blk, _):
        for k4 in range(4):
            v = dinvb[pl.ds(blk * 64 + k4 * 16, 16)]
            for lane in range(16):
                idx = jnp.full((16,), lane, jnp.int32)
                g = v.at[idx].get(mode="promise_in_bounds")
                for u in range(8):
                    rbuf[pl.ds((k4 * 16 + lane) * D + u * 16, 16)] = g
        pltpu.sync_copy(
            rbuf, dinv_hbm.at[pl.ds((node_base + blk * 64) * D, 64 * D)])
        return 0
    lax.fori_loop(0, NPT // 64, blk_body, 0)


@functools.partial(
    pl.kernel,
    out_type=jax.ShapeDtypeStruct((NACC * D,), jnp.float32),
    mesh=_mesh,
    scratch_types=[
        pltpu.VMEM((CPT, CH), jnp.int32),
        pltpu.VMEM((CH,), jnp.float32),
        pltpu.VMEM((CH,), jnp.float32),
        pltpu.VMEM((NPT,), jnp.float32),
        pltpu.VMEM((NPT,), jnp.float32),
        pltpu.VMEM((64 * D,), jnp.float32),
        pltpu.VMEM_SHARED((NACC,), jnp.float32),
        pltpu.SemaphoreType.DMA,
    ],
)
def _sc_deg(rowc_hbm, dinv_hbm, *rest):
    _sc_deg_kernel(rowc_hbm, dinv_hbm, *rest)


def _sc_seg_kernel(h_hbm, colc_hbm, rowc_hbm, z_hbm, out_hbm, colv, rowv,
                   rows0, rows1, acc_sh, sem0, sem1):
    cid = lax.axis_index("c")
    sid = lax.axis_index("s")
    wid = cid * NS + sid
    for k in range(RPT // 128):
        pltpu.sync_copy(z_hbm, acc_sh.at[pl.ds(sid * RPT + k * 128, 128)])
    plsc.subcore_barrier()

    bufs = (rows0, rows1)
    sems = (sem0, sem1)
    for p in range(CPW // HALF):
        pltpu.sync_copy(colc_hbm.at[pl.ds(wid * CPW + p * HALF, HALF)], colv)
        pltpu.sync_copy(rowc_hbm.at[pl.ds(wid * CPW + p * HALF, HALF)], rowv)
        for b in range(2):
            pltpu.async_copy(h_hbm.at[colv.at[b]], bufs[b], sems[b])

        def body(i, _):
            for b in range(2):
                j = i * 2 + b
                pltpu.make_async_copy(h_hbm.at[pl.ds(0, CH)], bufs[b],
                                      sems[b]).wait()
                pltpu.sync_copy(bufs[b], acc_sh.at[rowv.at[j]], add=True)

                @pl.when(j < HALF - 2)
                def _():
                    pltpu.async_copy(h_hbm.at[colv.at[j + 2]], bufs[b],
                                     sems[b])
            return 0
        lax.fori_loop(0, HALF // 2, body, 0)
    plsc.subcore_barrier()
    pltpu.sync_copy(acc_sh.at[pl.ds(sid * RPT, RPT)],
                    out_hbm.at[cid, pl.ds(sid * RPT, RPT)])


@functools.partial(
    pl.kernel,
    out_type=jax.ShapeDtypeStruct((NC, NACC, D), jnp.float32),
    mesh=_mesh,
    scratch_types=[
        pltpu.VMEM((HALF, CH), jnp.int32),
        pltpu.VMEM((HALF, CH), jnp.int32),
        pltpu.VMEM((CH, D), jnp.float32),
        pltpu.VMEM((CH, D), jnp.float32),
        pltpu.VMEM_SHARED((NACC, D), jnp.float32),
        pltpu.SemaphoreType.DMA,
        pltpu.SemaphoreType.DMA,
    ],
)
def _sc_seg(h_hbm, colc_hbm, rowc_hbm, z_hbm, *rest):
    _sc_seg_kernel(h_hbm, colc_hbm, rowc_hbm, z_hbm, *rest)


BLK = 2000


def _tc_in_kernel(x_ref, w_ref, b_ref, o_ref):
    o_ref[...] = lax.dot_general(
        x_ref[...], w_ref[...], (((1,), (1,)), ((), ())),
        preferred_element_type=jnp.float32) + b_ref[...]


def _tc_mid_kernel(p0_ref, p1_ref, dv_ref, w_ref, b_ref, o_ref):
    s = (p0_ref[0] + p1_ref[0]) * dv_ref[...]
    e = jnp.where(s > 0.0, s, jnp.exp(jnp.minimum(s, 0.0)) - 1.0)
    o_ref[...] = lax.dot_general(
        e, w_ref[...], (((1,), (1,)), ((), ())),
        preferred_element_type=jnp.float32) + b_ref[...]


def _tc_fin_kernel(p0_ref, p1_ref, dv_ref, o_ref):
    o_ref[...] = (p0_ref[0] + p1_ref[0]) * dv_ref[...]


def _tc_in(x, w, b2d):
    return pl.pallas_call(
        _tc_in_kernel,
        grid=(N_NODES // BLK,),
        in_specs=[
            pl.BlockSpec((BLK, D), lambda i: (i, 0)),
            pl.BlockSpec((D, D), lambda i: (0, 0)),
            pl.BlockSpec((1, D), lambda i: (0, 0)),
        ],
        out_specs=pl.BlockSpec((BLK, D), lambda i: (i, 0)),
        out_shape=jax.ShapeDtypeStruct((N_NODES, D), jnp.float32),
    )(x, w, b2d)


def _tc_mid(p, dinv, w, b2d):
    return pl.pallas_call(
        _tc_mid_kernel,
        grid=(N_NODES // BLK,),
        in_specs=[
            pl.BlockSpec((1, BLK, D), lambda i: (0, i, 0)),
            pl.BlockSpec((1, BLK, D), lambda i: (1, i, 0)),
            pl.BlockSpec((BLK, D), lambda i: (i, 0)),
            pl.BlockSpec((D, D), lambda i: (0, 0)),
            pl.BlockSpec((1, D), lambda i: (0, 0)),
        ],
        out_specs=pl.BlockSpec((BLK, D), lambda i: (i, 0)),
        out_shape=jax.ShapeDtypeStruct((N_NODES, D), jnp.float32),
    )(p, p, dinv, w, b2d)


def _tc_fin(p, dinv):
    return pl.pallas_call(
        _tc_fin_kernel,
        grid=(N_NODES // BLK,),
        in_specs=[
            pl.BlockSpec((1, BLK, D), lambda i: (0, i, 0)),
            pl.BlockSpec((1, BLK, D), lambda i: (1, i, 0)),
            pl.BlockSpec((BLK, D), lambda i: (i, 0)),
        ],
        out_specs=pl.BlockSpec((BLK, D), lambda i: (i, 0)),
        out_shape=jax.ShapeDtypeStruct((N_NODES, D), jnp.float32),
    )(p, p, dinv)


def kernel(x, edge_index, W1, b1, W2, b2):
    row = edge_index[0].astype(jnp.int32)
    col = edge_index[1].astype(jnp.int32)
    pad = E_PAD - N_EDGES
    rowc = jnp.concatenate(
        [row, jnp.full((pad,), TRASH, jnp.int32)]).reshape(NCHUNK, CH)
    colc = jnp.concatenate(
        [col, jnp.zeros((pad,), jnp.int32)]).reshape(NCHUNK, CH)
    b1r = b1.reshape(1, D)
    b2r = b2.reshape(1, D)

    zblk = jnp.zeros((128, D), jnp.float32)

    dinv = _sc_deg(rowc).reshape(NACC, D)
    h1 = _tc_in(x, W1, b1r)
    p1 = _sc_seg(h1, colc, rowc, zblk)
    h2 = _tc_mid(p1, dinv, W2, b2r)
    p2 = _sc_seg(h2, colc, rowc, zblk)
    return _tc_fin(p2, dinv)


# speedup vs baseline: 3.4923x; 1.0121x over previous
import functools

import jax
import jax.numpy as jnp
from jax import lax
from jax.experimental import pallas as pl
from jax.experimental.pallas import tpu as pltpu
from jax.experimental.pallas import tpu_sc as plsc

N_NODES = 10000
N_EDGES = 320000
D = 128

NC = 2
NS = 16
NW = NC * NS
CH = 128
NCHUNK = 2560
E_PAD = NCHUNK * CH
CPW = NCHUNK // NW
HALF = CPW // 2
GCH = 64
NCH2 = E_PAD // GCH
CPW2 = NCH2 // NW
HALF2 = CPW2 // 2
CPT = NCHUNK // NS
NACC = 10240
TRASH = N_NODES
RPT = NACC // NS
NPT = NACC // NW

_mesh = plsc.VectorSubcoreMesh(core_axis_name="c", subcore_axis_name="s")


def _fill_zero_row(zrow):
    for u in range(8):
        zrow[pl.ds(u * 16, 16)] = jnp.zeros((16,), jnp.float32)


def _sc_deg_kernel(rowc_hbm, dinv_hbm, rowv, ones, zrow, degb, dinvb, rbuf,
                   deg_sh, sem):
    cid = lax.axis_index("c")
    sid = lax.axis_index("s")
    for u in range(8):
        ones[pl.ds(u * 16, 16)] = jnp.ones((16,), jnp.float32)
    _fill_zero_row(zrow)
    pltpu.sync_copy(rowc_hbm.at[pl.ds(sid * CPT, CPT)], rowv)
    for k in range(RPT // 128):
        pltpu.sync_copy(zrow, deg_sh.at[pl.ds(sid * RPT + k * 128, 128)])
    plsc.subcore_barrier()

    def body(j, _):
        pltpu.sync_copy(ones, deg_sh.at[rowv.at[j]], add=True)
        return 0
    lax.fori_loop(0, CPT, body, 0)
    plsc.subcore_barrier()

    node_base = cid * (NACC // NC) + sid * NPT
    pltpu.sync_copy(deg_sh.at[pl.ds(node_base, NPT)], degb)
    for k in range(NPT // 16):
        v = degb[pl.ds(k * 16, 16)]
        dinvb[pl.ds(k * 16, 16)] = jnp.where(
            v == 0.0, jnp.zeros((16,), jnp.float32), 1.0 / v)

    def blk_body(blk, _):
        for k4 in range(4):
            v = dinvb[pl.ds(blk * 64 + k4 * 16, 16)]
            for lane in range(16):
                idx = jnp.full((16,), lane, jnp.int32)
                g = v.at[idx].get(mode="promise_in_bounds")
                for u in range(8):
                    rbuf[pl.ds((k4 * 16 + lane) * D + u * 16, 16)] = g
        pltpu.sync_copy(
            rbuf, dinv_hbm.at[pl.ds((node_base + blk * 64) * D, 64 * D)])
        return 0
    lax.fori_loop(0, NPT // 64, blk_body, 0)


@functools.partial(
    pl.kernel,
    out_type=jax.ShapeDtypeStruct((NACC * D,), jnp.float32),
    mesh=_mesh,
    scratch_types=[
        pltpu.VMEM((CPT, CH), jnp.int32),
        pltpu.VMEM((CH,), jnp.float32),
        pltpu.VMEM((CH,), jnp.float32),
        pltpu.VMEM((NPT,), jnp.float32),
        pltpu.VMEM((NPT,), jnp.float32),
        pltpu.VMEM((64 * D,), jnp.float32),
        pltpu.VMEM_SHARED((NACC,), jnp.float32),
        pltpu.SemaphoreType.DMA,
    ],
)
def _sc_deg(rowc_hbm, dinv_hbm, *rest):
    _sc_deg_kernel(rowc_hbm, dinv_hbm, *rest)


def _sc_seg_kernel(h_hbm, colf_hbm, rowc_hbm, z_hbm, out_hbm, colv, rowv,
                   b0, b1, b2, b3, acc_sh, g0, g1, g2, g3, s0, s1, s2, s3):
    cid = lax.axis_index("c")
    sid = lax.axis_index("s")
    wid = cid * NS + sid
    for k in range(RPT // 128):
        pltpu.sync_copy(z_hbm, acc_sh.at[pl.ds(sid * RPT + k * 128, 128)])
    plsc.subcore_barrier()

    bufs = (b0, b1, b2, b3)
    gsem = (g0, g1, g2, g3)
    ssem = (s0, s1, s2, s3)
    for p in range(CPW2 // HALF2):
        pltpu.sync_copy(
            colf_hbm.at[pl.ds((wid * CPW2 + p * HALF2) * GCH, HALF2 * GCH)],
            colv)
        pltpu.sync_copy(rowc_hbm.at[pl.ds(wid * CPW2 + p * HALF2, HALF2)],
                        rowv)
        for b in range(2):
            pltpu.async_copy(h_hbm.at[colv.at[pl.ds(b * GCH, GCH)]],
                             bufs[b], gsem[b])

        def body(i, _):
            for b in range(4):
                j = i * 4 + b
                nb = (b + 2) % 4
                pltpu.make_async_copy(h_hbm.at[pl.ds(0, GCH)], bufs[b],
                                      gsem[b]).wait()
                pltpu.async_copy(bufs[b], acc_sh.at[rowv.at[j]], ssem[b],
                                 add=True)

                @pl.when(j >= 2)
                def _():
                    pltpu.make_async_copy(bufs[nb], acc_sh.at[pl.ds(0, GCH)],
                                          ssem[nb]).wait()

                @pl.when(j < HALF2 - 2)
                def _():
                    pltpu.async_copy(
                        h_hbm.at[colv.at[pl.ds((j + 2) * GCH, GCH)]],
                        bufs[nb], gsem[nb])
            return 0
        lax.fori_loop(0, HALF2 // 4, body, 0)
        for b in (2, 3):
            pltpu.make_async_copy(bufs[b], acc_sh.at[pl.ds(0, GCH)],
                                  ssem[b]).wait()
    plsc.subcore_barrier()
    pltpu.sync_copy(acc_sh.at[pl.ds(sid * RPT, RPT)],
                    out_hbm.at[cid, pl.ds(sid * RPT, RPT)])


@functools.partial(
    pl.kernel,
    out_type=jax.ShapeDtypeStruct((NC, NACC, D), jnp.float32),
    mesh=_mesh,
    scratch_types=[
        pltpu.VMEM((HALF2 * GCH,), jnp.int32),
        pltpu.VMEM((HALF2, GCH), jnp.int32),
        pltpu.VMEM((GCH, D), jnp.float32),
        pltpu.VMEM((GCH, D), jnp.float32),
        pltpu.VMEM((GCH, D), jnp.float32),
        pltpu.VMEM((GCH, D), jnp.float32),
        pltpu.VMEM_SHARED((NACC, D), jnp.float32),
        pltpu.SemaphoreType.DMA,
        pltpu.SemaphoreType.DMA,
        pltpu.SemaphoreType.DMA,
        pltpu.SemaphoreType.DMA,
        pltpu.SemaphoreType.DMA,
        pltpu.SemaphoreType.DMA,
        pltpu.SemaphoreType.DMA,
        pltpu.SemaphoreType.DMA,
    ],
)
def _sc_seg(h_hbm, colf_hbm, rowc_hbm, z_hbm, *rest):
    _sc_seg_kernel(h_hbm, colf_hbm, rowc_hbm, z_hbm, *rest)


BLK = 2000


def _tc_in_kernel(x_ref, w_ref, b_ref, o_ref):
    o_ref[...] = lax.dot_general(
        x_ref[...], w_ref[...], (((1,), (1,)), ((), ())),
        preferred_element_type=jnp.float32) + b_ref[...]


def _tc_mid_kernel(p0_ref, p1_ref, dv_ref, w_ref, b_ref, o_ref):
    s = (p0_ref[0] + p1_ref[0]) * dv_ref[...]
    e = jnp.where(s > 0.0, s, jnp.exp(jnp.minimum(s, 0.0)) - 1.0)
    o_ref[...] = lax.dot_general(
        e, w_ref[...], (((1,), (1,)), ((), ())),
        preferred_element_type=jnp.float32) + b_ref[...]


def _tc_fin_kernel(p0_ref, p1_ref, dv_ref, o_ref):
    o_ref[...] = (p0_ref[0] + p1_ref[0]) * dv_ref[...]


def _tc_in(x, w, b2d):
    return pl.pallas_call(
        _tc_in_kernel,
        grid=(N_NODES // BLK,),
        in_specs=[
            pl.BlockSpec((BLK, D), lambda i: (i, 0)),
            pl.BlockSpec((D, D), lambda i: (0, 0)),
            pl.BlockSpec((1, D), lambda i: (0, 0)),
        ],
        out_specs=pl.BlockSpec((BLK, D), lambda i: (i, 0)),
        out_shape=jax.ShapeDtypeStruct((N_NODES, D), jnp.float32),
    )(x, w, b2d)


def _tc_mid(p, dinv, w, b2d):
    return pl.pallas_call(
        _tc_mid_kernel,
        grid=(N_NODES // BLK,),
        in_specs=[
            pl.BlockSpec((1, BLK, D), lambda i: (0, i, 0)),
            pl.BlockSpec((1, BLK, D), lambda i: (1, i, 0)),
            pl.BlockSpec((BLK, D), lambda i: (i, 0)),
            pl.BlockSpec((D, D), lambda i: (0, 0)),
            pl.BlockSpec((1, D), lambda i: (0, 0)),
        ],
        out_specs=pl.BlockSpec((BLK, D), lambda i: (i, 0)),
        out_shape=jax.ShapeDtypeStruct((N_NODES, D), jnp.float32),
    )(p, p, dinv, w, b2d)


def _tc_fin(p, dinv):
    return pl.pallas_call(
        _tc_fin_kernel,
        grid=(N_NODES // BLK,),
        in_specs=[
            pl.BlockSpec((1, BLK, D), lambda i: (0, i, 0)),
            pl.BlockSpec((1, BLK, D), lambda i: (1, i, 0)),
            pl.BlockSpec((BLK, D), lambda i: (i, 0)),
        ],
        out_specs=pl.BlockSpec((BLK, D), lambda i: (i, 0)),
        out_shape=jax.ShapeDtypeStruct((N_NODES, D), jnp.float32),
    )(p, p, dinv)


def kernel(x, edge_index, W1, b1, W2, b2):
    row = edge_index[0].astype(jnp.int32)
    col = edge_index[1].astype(jnp.int32)
    pad = E_PAD - N_EDGES
    trash = TRASH + jnp.arange(pad, dtype=jnp.int32) % (NACC - N_NODES)
    rowc = jnp.concatenate([row, trash]).reshape(NCHUNK, CH)
    colc = jnp.concatenate(
        [col, jnp.zeros((pad,), jnp.int32)]).reshape(NCHUNK, CH)
    b1r = b1.reshape(1, D)
    b2r = b2.reshape(1, D)

    zblk = jnp.zeros((128, D), jnp.float32)
    colf = colc.reshape(-1)
    rowc2 = rowc.reshape(NCH2, GCH)

    dinv = _sc_deg(rowc).reshape(NACC, D)
    h1 = _tc_in(x, W1, b1r)
    p1 = _sc_seg(h1, colf, rowc2, zblk)
    h2 = _tc_mid(p1, dinv, W2, b2r)
    p2 = _sc_seg(h2, colf, rowc2, zblk)
    return _tc_fin(p2, dinv)
